# baseline (speedup 1.0000x reference)
import functools

import jax
import jax.numpy as jnp
from jax.experimental import pallas as pl
from jax.experimental.pallas import tpu as pltpu

H = 12
_INTERPRET = False


def _qkv_proj_kernel(x_ref, w_ref, b_ref, o_ref):
    x = x_ref[...]
    for j in range(w_ref.shape[0]):
        o_ref[j] = (
            jnp.dot(x, w_ref[j], preferred_element_type=jnp.float32) + b_ref[j]
        )


def _attn_kernel(q_ref, k_ref, v_ref, o_ref, *, scale):
    q = q_ref[0]
    k = k_ref[0]
    v = v_ref[0]
    s = jax.lax.dot_general(
        q, k, (((1,), (1,)), ((), ())), preferred_element_type=jnp.float32
    ) * scale
    m = jnp.max(s, axis=-1, keepdims=True)
    p = jnp.exp(s - m)
    l = jnp.sum(p, axis=-1, keepdims=True)
    o_ref[0] = jnp.dot(p, v, preferred_element_type=jnp.float32) / l


def _ln(y, g, b):
    mu = jnp.mean(y, axis=-1, keepdims=True)
    d = y - mu
    var = jnp.mean(d * d, axis=-1, keepdims=True)
    return d * jax.lax.rsqrt(var + 1e-5) * g + b


def _oproj_ln_kernel(a_ref, w_ref, b_ref, res_ref, g_ref, be_ref, out_ref):
    y = b_ref[...] + res_ref[...]
    for h in range(w_ref.shape[0]):
        y = y + jnp.dot(a_ref[h], w_ref[h], preferred_element_type=jnp.float32)
    out_ref[...] = _ln(y, g_ref[...], be_ref[...])


def _router_kernel(x_ref, wg_ref, bg_ref, gw_ref, usage_ref, *, n_exp):
    i = pl.program_id(0)
    scores = (
        jnp.dot(x_ref[...], wg_ref[...], preferred_element_type=jnp.float32)
        + bg_ref[...]
    )
    cols = jax.lax.broadcasted_iota(jnp.int32, scores.shape, 1)
    m1 = jnp.max(scores, axis=-1, keepdims=True)
    a1 = jnp.min(jnp.where(scores == m1, cols, n_exp), axis=-1, keepdims=True)
    masked = jnp.where(cols == a1, -jnp.inf, scores)
    m2 = jnp.max(masked, axis=-1, keepdims=True)
    a2 = jnp.min(jnp.where(masked == m2, cols, n_exp), axis=-1, keepdims=True)
    e2 = jnp.exp(m2 - m1)
    p1 = 1.0 / (1.0 + e2)
    p2 = e2 * p1
    gw = jnp.where(cols == a1, p1, 0.0) + jnp.where(cols == a2, p2, 0.0)
    gw_ref[...] = gw

    @pl.when(i == 0)
    def _():
        usage_ref[...] = jnp.zeros_like(usage_ref)

    usage_ref[...] += jnp.sum(gw, axis=0, keepdims=True)


def _moe_dense_kernel(
    x_ref, w1_ref, b1_ref, w2_ref, b2_ref, gw_ref, g_ref, be_ref,
    out_ref, acc_ref, *, n_exp, sblk,
):
    e = pl.program_id(0)
    s = pl.program_id(1)
    h = jnp.maximum(
        jnp.dot(x_ref[...], w1_ref[0], preferred_element_type=jnp.float32)
        + b1_ref[0],
        0.0,
    )
    cols = jax.lax.broadcasted_iota(jnp.int32, gw_ref.shape, 1)
    gwe = jnp.sum(jnp.where(cols == e, gw_ref[...], 0.0), axis=1, keepdims=True)
    y = (
        jnp.dot(h, w2_ref[0], preferred_element_type=jnp.float32)
        + b2_ref[0]
    ) * gwe
    rows = pl.ds(s * sblk, sblk)

    @pl.when(e == 0)
    def _():
        acc_ref[rows, :] = y

    @pl.when(e > 0)
    def _():
        acc_ref[rows, :] += y

    @pl.when(e == n_exp - 1)
    def _():
        out_ref[...] = _ln(x_ref[...] + acc_ref[rows, :], g_ref[...], be_ref[...])


def _aux_kernel(u_ref, aux_ref, *, n_layers):
    u = u_ref[...]
    p = u / jnp.sum(u, axis=-1, keepdims=True)
    ent = -jnp.sum(p * jnp.log(p + 1e-9), axis=-1)
    aux_ref[...] = (jnp.sum(ent) / n_layers).reshape(1, 1)


def _encoder_layer(x, w3, b3, wo3, bo, g1, be1, g2, be2, wg_t, bg,
                   w1_t, b1, w2_t, b2):
    S, D = x.shape
    E, _, DFF = w1_t.shape
    hd = D // H
    sblk = 256 if S % 256 == 0 else S
    nsb = S // sblk

    qkv = pl.pallas_call(
        _qkv_proj_kernel,
        grid=(nsb,),
        in_specs=[
            pl.BlockSpec((sblk, D), lambda i: (i, 0)),
            pl.BlockSpec((3 * H, D, hd), lambda i: (0, 0, 0)),
            pl.BlockSpec((3 * H, 1, hd), lambda i: (0, 0, 0)),
        ],
        out_specs=pl.BlockSpec((3 * H, sblk, hd), lambda i: (0, i, 0)),
        out_shape=jax.ShapeDtypeStruct((3 * H, S, hd), jnp.float32),
        interpret=_INTERPRET,
    )(x, w3, b3)

    attn = pl.pallas_call(
        functools.partial(_attn_kernel, scale=1.0 / float(hd) ** 0.5),
        grid=(H, nsb),
        in_specs=[
            pl.BlockSpec((1, sblk, hd), lambda h, i: (h, i, 0)),
            pl.BlockSpec((1, S, hd), lambda h, i: (H + h, 0, 0)),
            pl.BlockSpec((1, S, hd), lambda h, i: (2 * H + h, 0, 0)),
        ],
        out_specs=pl.BlockSpec((1, sblk, hd), lambda h, i: (h, i, 0)),
        out_shape=jax.ShapeDtypeStruct((H, S, hd), jnp.float32),
        interpret=_INTERPRET,
    )(qkv, qkv, qkv)

    x1 = pl.pallas_call(
        _oproj_ln_kernel,
        grid=(nsb,),
        in_specs=[
            pl.BlockSpec((H, sblk, hd), lambda i: (0, i, 0)),
            pl.BlockSpec((H, hd, D), lambda i: (0, 0, 0)),
            pl.BlockSpec((1, D), lambda i: (0, 0)),
            pl.BlockSpec((sblk, D), lambda i: (i, 0)),
            pl.BlockSpec((1, D), lambda i: (0, 0)),
            pl.BlockSpec((1, D), lambda i: (0, 0)),
        ],
        out_specs=pl.BlockSpec((sblk, D), lambda i: (i, 0)),
        out_shape=jax.ShapeDtypeStruct((S, D), jnp.float32),
        interpret=_INTERPRET,
    )(attn, wo3, bo.reshape(1, D), x, g1.reshape(1, D), be1.reshape(1, D))

    gw, usage = pl.pallas_call(
        functools.partial(_router_kernel, n_exp=E),
        grid=(nsb,),
        in_specs=[
            pl.BlockSpec((sblk, D), lambda i: (i, 0)),
            pl.BlockSpec((D, E), lambda i: (0, 0)),
            pl.BlockSpec((1, E), lambda i: (0, 0)),
        ],
        out_specs=[
            pl.BlockSpec((sblk, E), lambda i: (i, 0)),
            pl.BlockSpec((1, E), lambda i: (0, 0)),
        ],
        out_shape=[
            jax.ShapeDtypeStruct((S, E), jnp.float32),
            jax.ShapeDtypeStruct((1, E), jnp.float32),
        ],
        interpret=_INTERPRET,
    )(x1, wg_t, bg.reshape(1, E))

    x2 = pl.pallas_call(
        functools.partial(_moe_dense_kernel, n_exp=E, sblk=sblk),
        grid=(E, nsb),
        in_specs=[
            pl.BlockSpec((sblk, D), lambda e, i: (i, 0)),
            pl.BlockSpec((1, D, DFF), lambda e, i: (e, 0, 0)),
            pl.BlockSpec((1, 1, DFF), lambda e, i: (e, 0, 0)),
            pl.BlockSpec((1, DFF, D), lambda e, i: (e, 0, 0)),
            pl.BlockSpec((1, 1, D), lambda e, i: (e, 0, 0)),
            pl.BlockSpec((sblk, E), lambda e, i: (i, 0)),
            pl.BlockSpec((1, D), lambda e, i: (0, 0)),
            pl.BlockSpec((1, D), lambda e, i: (0, 0)),
        ],
        out_specs=pl.BlockSpec((sblk, D), lambda e, i: (i, 0)),
        out_shape=jax.ShapeDtypeStruct((S, D), jnp.float32),
        scratch_shapes=[pltpu.VMEM((S, D), jnp.float32)],
        interpret=_INTERPRET,
    )(x1, w1_t, b1.reshape(E, 1, DFF), w2_t, b2.reshape(E, 1, D), gw,
      g2.reshape(1, D), be2.reshape(1, D))

    return x2, usage


def kernel(src, Wqkv, bqkv, Wo, bo, g1, be1, g2, be2, Wg, bg, W1, b1, W2, b2):
    L = Wqkv.shape[0]
    S, B, D = src.shape
    hd = D // H
    x = src.reshape(S * B, D)
    usages = []
    for l in range(L):
        x, usage = _encoder_layer(
            x,
            jnp.swapaxes(Wqkv[l].reshape(3 * H, hd, D), 1, 2),
            bqkv[l].reshape(3 * H, 1, hd),
            Wo[l].T.reshape(H, hd, D),
            bo[l], g1[l], be1[l], g2[l], be2[l],
            Wg[l].T, bg[l], jnp.swapaxes(W1[l], 1, 2), b1[l],
            jnp.swapaxes(W2[l], 1, 2), b2[l],
        )
        usages.append(usage)

    usage_all = jnp.concatenate(usages, axis=0)
    aux = pl.pallas_call(
        functools.partial(_aux_kernel, n_layers=L),
        out_shape=jax.ShapeDtypeStruct((1, 1), jnp.float32),
        interpret=_INTERPRET,
    )(usage_all)
    return x.reshape(S, B, D), aux.reshape(())
